# baseline (speedup 1.0000x reference)
import functools

import jax
import jax.numpy as jnp
from jax import lax
from jax.experimental import pallas as pl
from jax.experimental.pallas import tpu as pltpu
from jax.experimental.pallas import tpu_sc as plsc

NUM_CORES = 2
NUM_SUBCORES = 16
NW = NUM_CORES * NUM_SUBCORES
CHUNK = 128


def _emb_kernel(n_tokens, seq_len, dim):
    n_per_w = n_tokens // NW
    k = n_per_w // CHUNK
    mesh = plsc.VectorSubcoreMesh(core_axis_name="c", subcore_axis_name="s")

    @functools.partial(
        pl.kernel,
        mesh=mesh,
        out_type=jax.ShapeDtypeStruct((n_tokens, dim), jnp.float32),
        scratch_types=[
            pltpu.VMEM((k, CHUNK), jnp.int32),
            pltpu.VMEM((n_per_w, dim), jnp.float32),
            pltpu.SemaphoreType.DMA((k,)),
            pltpu.SemaphoreType.DMA,
            pltpu.SemaphoreType.DMA((k,)),
            pltpu.SemaphoreType.DMA,
        ],
    )
    def emb(idx_hbm, table_hbm, pos_hbm, out_hbm, idx_v, rows_v, sem_p, sem_i,
            sem_g, sem_o):
        wid = lax.axis_index("s") * NUM_CORES + lax.axis_index("c")
        base = wid * n_per_w
        pos_base = lax.rem(base, seq_len)
        pos_cps = [
            pltpu.async_copy(pos_hbm.at[pl.ds(pos_base + j * CHUNK, CHUNK)],
                             rows_v.at[pl.ds(j * CHUNK, CHUNK)], sem_p.at[j])
            for j in range(k)
        ]
        cp_idx = pltpu.async_copy(idx_hbm.at[pl.ds(wid * k, k)], idx_v, sem_i)
        cp_idx.wait()
        gathers = []
        for j in range(k):
            pos_cps[j].wait()
            gathers.append(
                pltpu.async_copy(table_hbm.at[idx_v.at[j]],
                                 rows_v.at[pl.ds(j * CHUNK, CHUNK)],
                                 sem_g.at[j], add=True))
        outs = []
        for j in range(k):
            gathers[j].wait()
            outs.append(
                pltpu.async_copy(rows_v.at[pl.ds(j * CHUNK, CHUNK)],
                                 out_hbm.at[pl.ds(base + j * CHUNK, CHUNK)],
                                 sem_o))
        for c in outs:
            c.wait()

    return emb


def kernel(inputs, W_word, P_pos):
    batch, seq_len = inputs.shape
    vocab, dim = W_word.shape
    n_tokens = batch * seq_len
    idx2d = inputs.reshape(n_tokens // CHUNK, CHUNK)
    out = _emb_kernel(n_tokens, seq_len, dim)(idx2d, W_word, P_pos)
    return out.reshape(batch, seq_len, dim)


# speedup vs baseline: 1.0093x; 1.0093x over previous
import functools

import jax
import jax.numpy as jnp
from jax import lax
from jax.experimental import pallas as pl
from jax.experimental.pallas import tpu as pltpu
from jax.experimental.pallas import tpu_sc as plsc

NUM_CORES = 2
NUM_SUBCORES = 16
NW = NUM_CORES * NUM_SUBCORES
CHUNK = 128


def _emb_kernel(n_tokens, seq_len, dim):
    n_per_w = n_tokens // NW
    k = n_per_w // CHUNK
    mesh = plsc.VectorSubcoreMesh(core_axis_name="c", subcore_axis_name="s")

    @functools.partial(
        pl.kernel,
        mesh=mesh,
        out_type=jax.ShapeDtypeStruct((n_tokens, dim), jnp.float32),
        scratch_types=[
            pltpu.VMEM((k, CHUNK), jnp.int32),
            pltpu.VMEM((n_per_w, dim), jnp.float32),
            pltpu.SemaphoreType.DMA((k,)),
            pltpu.SemaphoreType.DMA,
            pltpu.SemaphoreType.DMA((k,)),
            pltpu.SemaphoreType.DMA,
        ],
    )
    def emb(idx_hbm, table_hbm, pos_hbm, out_hbm, idx_v, rows_v, sem_p, sem_i,
            sem_g, sem_o):
        wid = lax.axis_index("s") * NUM_CORES + lax.axis_index("c")
        base = wid * n_per_w
        pos_base = lax.rem(base, seq_len)
        b_row = lax.div(base, seq_len)
        pos_cps = [
            pltpu.async_copy(pos_hbm.at[pl.ds(pos_base + j * CHUNK, CHUNK)],
                             rows_v.at[pl.ds(j * CHUNK, CHUNK)], sem_p.at[j])
            for j in range(k)
        ]
        idx_cps = [
            pltpu.async_copy(idx_hbm.at[b_row, pl.ds(pos_base + j * CHUNK, CHUNK)],
                             idx_v.at[j], sem_i)
            for j in range(k)
        ]
        for c in idx_cps:
            c.wait()
        gathers = []
        for j in range(k):
            pos_cps[j].wait()
            gathers.append(
                pltpu.async_copy(table_hbm.at[idx_v.at[j]],
                                 rows_v.at[pl.ds(j * CHUNK, CHUNK)],
                                 sem_g.at[j], add=True))
        outs = []
        for j in range(k):
            gathers[j].wait()
            outs.append(
                pltpu.async_copy(rows_v.at[pl.ds(j * CHUNK, CHUNK)],
                                 out_hbm.at[pl.ds(base + j * CHUNK, CHUNK)],
                                 sem_o))
        for c in outs:
            c.wait()

    return emb


def kernel(inputs, W_word, P_pos):
    batch, seq_len = inputs.shape
    vocab, dim = W_word.shape
    n_tokens = batch * seq_len
    out = _emb_kernel(n_tokens, seq_len, dim)(inputs, W_word, P_pos)
    return out.reshape(batch, seq_len, dim)


# speedup vs baseline: 1.0160x; 1.0066x over previous
import functools

import jax
import jax.numpy as jnp
from jax import lax
from jax.experimental import pallas as pl
from jax.experimental.pallas import tpu as pltpu
from jax.experimental.pallas import tpu_sc as plsc

NUM_CORES = 2
NUM_SUBCORES = 16
NW = NUM_CORES * NUM_SUBCORES
CHUNK = 128


def _emb_kernel(n_tokens, seq_len, dim):
    n_per_w = n_tokens // NW
    k = n_per_w // CHUNK
    mesh = plsc.VectorSubcoreMesh(core_axis_name="c", subcore_axis_name="s")

    @functools.partial(
        pl.kernel,
        mesh=mesh,
        out_type=jax.ShapeDtypeStruct((n_tokens // seq_len, seq_len, dim), jnp.float32),
        scratch_types=[
            pltpu.VMEM((k, CHUNK), jnp.int32),
            pltpu.VMEM((n_per_w, dim), jnp.float32),
            pltpu.SemaphoreType.DMA((k,)),
            pltpu.SemaphoreType.DMA,
            pltpu.SemaphoreType.DMA((k,)),
            pltpu.SemaphoreType.DMA,
        ],
    )
    def emb(idx_hbm, table_hbm, pos_hbm, out_hbm, idx_v, rows_v, sem_p, sem_i,
            sem_g, sem_o):
        wid = lax.axis_index("s") * NUM_CORES + lax.axis_index("c")
        base = wid * n_per_w
        pos_base = lax.rem(base, seq_len)
        b_row = lax.div(base, seq_len)
        pos_cps = [
            pltpu.async_copy(pos_hbm.at[pl.ds(pos_base + j * CHUNK, CHUNK)],
                             rows_v.at[pl.ds(j * CHUNK, CHUNK)], sem_p.at[j])
            for j in range(k)
        ]
        idx_cps = [
            pltpu.async_copy(idx_hbm.at[b_row, pl.ds(pos_base + j * CHUNK, CHUNK)],
                             idx_v.at[j], sem_i)
            for j in range(k)
        ]
        for c in idx_cps:
            c.wait()
        gathers = []
        for j in range(k):
            pos_cps[j].wait()
            gathers.append(
                pltpu.async_copy(table_hbm.at[idx_v.at[j]],
                                 rows_v.at[pl.ds(j * CHUNK, CHUNK)],
                                 sem_g.at[j], add=True))
        outs = []
        for j in range(k):
            gathers[j].wait()
            outs.append(
                pltpu.async_copy(rows_v.at[pl.ds(j * CHUNK, CHUNK)],
                                 out_hbm.at[b_row, pl.ds(pos_base + j * CHUNK, CHUNK)],
                                 sem_o))
        for c in outs:
            c.wait()

    return emb


def kernel(inputs, W_word, P_pos):
    batch, seq_len = inputs.shape
    vocab, dim = W_word.shape
    n_tokens = batch * seq_len
    return _emb_kernel(n_tokens, seq_len, dim)(inputs, W_word, P_pos)
